# baseline (speedup 1.0000x reference)
import functools

import jax
import jax.numpy as jnp
from jax import lax
from jax.experimental import pallas as pl
from jax.experimental.pallas import tpu as pltpu
from jax.experimental.pallas import tpu_sc as plsc

_NBUF = 4
_H = 524288


@functools.lru_cache(maxsize=None)
def _make_tc_transpose(V, E, BK, H):
    nin = pl.cdiv(V, BK)

    def body(lo_ref, hi_ref, o_ref):
        o_ref[:, :E] = jnp.transpose(lo_ref[...], (1, 0))
        o_ref[:, E:] = jnp.transpose(hi_ref[...], (1, 0))

    return pl.pallas_call(
        body,
        grid=(H // BK,),
        in_specs=[
            pl.BlockSpec((E, BK), lambda i: (0, i)),
            pl.BlockSpec((E, BK),
                         lambda i: (0, jnp.minimum(i + H // BK, nin - 1))),
        ],
        out_specs=pl.BlockSpec((BK, 2 * E), lambda i: (i, 0)),
        out_shape=jax.ShapeDtypeStruct((H, 2 * E), jnp.float32),
    )


@functools.lru_cache(maxsize=None)
def _make_sc_gather(NW, NCHUNK, C, E, V):
    mesh = plsc.VectorSubcoreMesh(core_axis_name="c", subcore_axis_name="s")
    NC = 2
    N = NW * NCHUNK * C

    @functools.partial(
        pl.kernel,
        mesh=mesh,
        compiler_params=pltpu.CompilerParams(use_tc_tiling_on_sc=False),
        out_type=jax.ShapeDtypeStruct((N, 2 * E), jnp.float32),
        scratch_types=(
            [pltpu.VMEM((C,), jnp.int32) for _ in range(_NBUF)]
            + [pltpu.VMEM((C, 2 * E), jnp.float32) for _ in range(_NBUF)]
            + [pltpu.SemaphoreType.DMA] * (3 * _NBUF)
        ),
    )
    def sc_call(x_hbm, tok_hbm, out_hbm, *scratch):
        idxs = scratch[:_NBUF]
        bufs = scratch[_NBUF:2 * _NBUF]
        igs = scratch[2 * _NBUF:3 * _NBUF]
        gs = scratch[3 * _NBUF:4 * _NBUF]
        ss = scratch[4 * _NBUF:5 * _NBUF]

        wid = lax.axis_index("s") * NC + lax.axis_index("c")
        base = wid * (NCHUNK * C)

        def idx_copy(c):
            return pltpu.async_copy(
                x_hbm.at[pl.ds(base + c * C, C)], idxs[c % _NBUF],
                igs[c % _NBUF])

        def gather(c):
            return pltpu.async_copy(
                tok_hbm.at[idxs[c % _NBUF]], bufs[c % _NBUF], gs[c % _NBUF])

        def scatter(c):
            return pltpu.async_copy(
                bufs[c % _NBUF], out_hbm.at[pl.ds(base + c * C, C)],
                ss[c % _NBUF])

        icopies = [None] * NCHUNK
        gathers = [None] * NCHUNK
        scatters = [None] * NCHUNK

        for k in range(min(_NBUF, NCHUNK)):
            icopies[k] = idx_copy(k)
        for k in range(min(_NBUF - 1, NCHUNK)):
            icopies[k].wait()
            gathers[k] = gather(k)

        for c in range(NCHUNK):
            gathers[c].wait()
            if c + _NBUF < NCHUNK:
                icopies[c + _NBUF] = idx_copy(c + _NBUF)
            if c + _NBUF - 1 < NCHUNK:
                icopies[c + _NBUF - 1].wait()
                if c >= 1:
                    scatters[c - 1].wait()
                gathers[c + _NBUF - 1] = gather(c + _NBUF - 1)
            scatters[c] = scatter(c)

        for k in range(max(0, NCHUNK - _NBUF), NCHUNK):
            if scatters[k] is not None:
                scatters[k].wait()

    return sc_call


@functools.lru_cache(maxsize=None)
def _make_tc_add(B, S, E, SB):
    def body(g_ref, x_ref, p_ref, o_ref):
        for s in range(SB):
            p = p_ref[s:s + 1, :]
            g = g_ref[s] + jnp.concatenate([p, p], axis=1)
            lo = jnp.transpose(g[:, :E], (1, 0))
            hi = jnp.transpose(g[:, E:], (1, 0))
            m = x_ref[s:s + 1, :] >= _H
            o_ref[s] = jnp.where(m, hi, lo)

    return pl.pallas_call(
        body,
        grid=(S // SB,),
        in_specs=[
            pl.BlockSpec((SB, B, 2 * E), lambda i: (i, 0, 0)),
            pl.BlockSpec((SB, B), lambda i: (i, 0)),
            pl.BlockSpec((SB, E), lambda i: (i, 0)),
        ],
        out_specs=pl.BlockSpec((SB, E, B), lambda i: (i, 0, 0)),
        out_shape=jax.ShapeDtypeStruct((S, E, B), jnp.float32),
    )


def kernel(x, token_table, pos_table):
    B, S = x.shape
    V, E = token_table.shape
    N = B * S

    NW = 32
    C = 200
    NCHUNK = N // (NW * C)
    assert NW * NCHUNK * C == N and C % 8 == 0 and E % 16 == 0 and V % 2 == 0

    assert V <= 2 * _H and _H % 8192 == 0
    xt = x.T
    xtf = xt.reshape(N).astype(jnp.int32)
    xg = jnp.where(xtf >= _H, xtf - _H, xtf)
    tt = token_table.T
    tp = _make_tc_transpose(V, E, 16384, _H)(tt, tt)

    CH = 128
    splits = [(0, 96), (96, 200)]
    ots = []
    for s0, s1 in splits:
        S2 = s1 - s0
        gah = _make_sc_gather(NW, S2 * B // (NW * CH), CH, E, V)(
            xg[s0 * B:s1 * B], tp)
        ots.append(_make_tc_add(B, S2, E, 8)(
            gah.reshape(S2, B, 2 * E), xt[s0:s1], pos_table[s0:s1]))
    return jnp.transpose(jnp.concatenate(ots, axis=0), (2, 0, 1))


# speedup vs baseline: 1.0466x; 1.0466x over previous
import functools

import jax
import jax.numpy as jnp
from jax import lax
from jax.experimental import pallas as pl
from jax.experimental.pallas import tpu as pltpu
from jax.experimental.pallas import tpu_sc as plsc

_NBUF = 4
_H = 524288


@functools.lru_cache(maxsize=None)
def _make_tc_transpose(V, E, BK, H):
    nin = pl.cdiv(V, BK)

    def body(lo_ref, hi_ref, o_ref):
        o_ref[:, :E] = jnp.transpose(lo_ref[...], (1, 0))
        o_ref[:, E:] = jnp.transpose(hi_ref[...], (1, 0))

    return pl.pallas_call(
        body,
        grid=(H // BK,),
        in_specs=[
            pl.BlockSpec((E, BK), lambda i: (0, i)),
            pl.BlockSpec((E, BK),
                         lambda i: (0, jnp.minimum(i + H // BK, nin - 1))),
        ],
        out_specs=pl.BlockSpec((BK, 2 * E), lambda i: (i, 0)),
        out_shape=jax.ShapeDtypeStruct((H, 2 * E), jnp.float32),
    )


@functools.lru_cache(maxsize=None)
def _make_sc_gather(NW, NCHUNK, C, E, V):
    mesh = plsc.VectorSubcoreMesh(core_axis_name="c", subcore_axis_name="s")
    NC = 2
    N = NW * NCHUNK * C

    @functools.partial(
        pl.kernel,
        mesh=mesh,
        compiler_params=pltpu.CompilerParams(use_tc_tiling_on_sc=False),
        out_type=jax.ShapeDtypeStruct((N, 2 * E), jnp.float32),
        scratch_types=(
            [pltpu.VMEM((C,), jnp.int32) for _ in range(_NBUF)]
            + [pltpu.VMEM((C, 2 * E), jnp.float32) for _ in range(_NBUF)]
            + [pltpu.SemaphoreType.DMA] * (3 * _NBUF)
        ),
    )
    def sc_call(x_hbm, tok_hbm, out_hbm, *scratch):
        idxs = scratch[:_NBUF]
        bufs = scratch[_NBUF:2 * _NBUF]
        igs = scratch[2 * _NBUF:3 * _NBUF]
        gs = scratch[3 * _NBUF:4 * _NBUF]
        ss = scratch[4 * _NBUF:5 * _NBUF]

        wid = lax.axis_index("s") * NC + lax.axis_index("c")
        base = wid * (NCHUNK * C)

        def idx_copy(c):
            return pltpu.async_copy(
                x_hbm.at[pl.ds(base + c * C, C)], idxs[c % _NBUF],
                igs[c % _NBUF])

        def gather(c):
            return pltpu.async_copy(
                tok_hbm.at[idxs[c % _NBUF]], bufs[c % _NBUF], gs[c % _NBUF])

        def scatter(c):
            return pltpu.async_copy(
                bufs[c % _NBUF], out_hbm.at[pl.ds(base + c * C, C)],
                ss[c % _NBUF])

        icopies = [None] * NCHUNK
        gathers = [None] * NCHUNK
        scatters = [None] * NCHUNK

        for k in range(min(_NBUF, NCHUNK)):
            icopies[k] = idx_copy(k)
        for k in range(min(_NBUF - 1, NCHUNK)):
            icopies[k].wait()
            gathers[k] = gather(k)

        for c in range(NCHUNK):
            gathers[c].wait()
            if c + _NBUF < NCHUNK:
                icopies[c + _NBUF] = idx_copy(c + _NBUF)
            if c + _NBUF - 1 < NCHUNK:
                icopies[c + _NBUF - 1].wait()
                if c >= 1:
                    scatters[c - 1].wait()
                gathers[c + _NBUF - 1] = gather(c + _NBUF - 1)
            scatters[c] = scatter(c)

        for k in range(max(0, NCHUNK - _NBUF), NCHUNK):
            if scatters[k] is not None:
                scatters[k].wait()

    return sc_call


@functools.lru_cache(maxsize=None)
def _make_tc_add(B, S, E, SB):
    def body(g_ref, x_ref, p_ref, o_ref):
        for s in range(SB):
            p = p_ref[s:s + 1, :]
            g = g_ref[s] + jnp.concatenate([p, p], axis=1)
            lo = jnp.transpose(g[:, :E], (1, 0))
            hi = jnp.transpose(g[:, E:], (1, 0))
            m = x_ref[s:s + 1, :] >= _H
            o_ref[s] = jnp.where(m, hi, lo)

    return pl.pallas_call(
        body,
        grid=(S // SB,),
        in_specs=[
            pl.BlockSpec((SB, B, 2 * E), lambda i: (i, 0, 0)),
            pl.BlockSpec((SB, B), lambda i: (i, 0)),
            pl.BlockSpec((SB, E), lambda i: (i, 0)),
        ],
        out_specs=pl.BlockSpec((SB, E, B), lambda i: (i, 0, 0)),
        out_shape=jax.ShapeDtypeStruct((S, E, B), jnp.float32),
    )


def kernel(x, token_table, pos_table):
    B, S = x.shape
    V, E = token_table.shape
    N = B * S

    NW = 32
    C = 200
    NCHUNK = N // (NW * C)
    assert NW * NCHUNK * C == N and C % 8 == 0 and E % 16 == 0 and V % 2 == 0

    assert V <= 2 * _H and _H % 8192 == 0
    xt = x.T
    xtf = xt.reshape(N).astype(jnp.int32)
    xg = jnp.where(xtf >= _H, xtf - _H, xtf)
    tt = token_table.T
    tp = _make_tc_transpose(V, E, 16384, _H)(tt, tt)
    ga = _make_sc_gather(NW, NCHUNK, C, E, V)(xg, tp)
    ot = _make_tc_add(B, S, E, 8)(ga.reshape(S, B, 2 * E), xt, pos_table[:S])
    return jnp.transpose(ot, (2, 0, 1))
